# baseline (speedup 1.0000x reference)
import functools

import jax
import jax.numpy as jnp
from jax import lax
from jax.experimental import pallas as pl
from jax.experimental.pallas import tpu as pltpu
from jax.experimental.pallas import tpu_sc as plsc

_TCA = 25
_GCA = 9
_KS = (24, 23, 22)
_ITERS = 10
_MAXK = 24
_VOCAB = 1000000
_DIM = 32
_NSCAN = _VOCAB - 1
_NW = 32
_T16 = 31248
_TLAST = _VOCAB - (_NW - 1) * _T16
_CR = 1024
_NCH = 32
_GRP = _CR // 16
_BIG = 3.0e38


def _kmeans_run(data, pmask, k):
    jj = lax.broadcasted_iota(jnp.int32, (1, _MAXK), 1)
    kmask = jj < k
    cents0 = data[:_MAXK]
    ones_col = jnp.ones((32, 1), jnp.float32)

    def labels_of(cents):
        diff = data[:, None, :] - cents[None, :, :]
        d2 = jnp.sum(diff * diff, axis=-1)
        d2 = jnp.where(kmask, d2, _BIG)
        m = jnp.min(d2, axis=1, keepdims=True)
        return jnp.min(jnp.where(d2 == m, jj, _MAXK), axis=1, keepdims=True)

    def body(_, cents):
        lab = labels_of(cents)
        onehot = ((lab == jj) & pmask).astype(jnp.float32)
        counts = lax.dot_general(onehot, ones_col,
                                 (((0,), (0,)), ((), ())))
        sums = lax.dot_general(onehot, data,
                               (((0,), (0,)), ((), ())))
        newc = sums / jnp.maximum(counts, 1.0)
        return jnp.where(counts > 0, newc, cents)

    cents = lax.fori_loop(0, _ITERS, body, cents0)
    lab = labels_of(cents)
    onehot = ((lab == jj) & pmask).astype(jnp.float32)
    ii = lax.broadcasted_iota(jnp.int32, (32, 1), 0)
    good = jnp.sum(onehot * (ii < _GCA), axis=0, keepdims=True)
    bad = jnp.sum(onehot * ((ii >= _GCA) & pmask), axis=0, keepdims=True)
    sizes = jnp.where((bad == 0.0) & (good > 0.0), good, 0.0)
    return sizes, cents


def _kmeans_centroid(data, pmask):
    jj = lax.broadcasted_iota(jnp.int32, (1, _MAXK), 1)
    ms, cents_sel = [], []
    for k in _KS:
        sizes, cents = _kmeans_run(data, pmask, k)
        m = jnp.max(sizes)
        arg = jnp.min(jnp.where(sizes == m, jj, _MAXK))
        oh = (jj == arg).astype(jnp.float32)
        csel = lax.dot_general(oh, cents, (((1,), (0,)), ((), ())))
        ms.append(m)
        cents_sel.append(csel)

    gm = jnp.maximum(jnp.maximum(ms[0], ms[1]), ms[2])
    s0 = ms[0] == gm
    s1 = (ms[1] == gm) & (~s0)
    s2 = (ms[2] == gm) & (~s0) & (~s1)
    centroid = (jnp.where(s0, 1.0, 0.0) * cents_sel[0]
                + jnp.where(s1, 1.0, 0.0) * cents_sel[1]
                + jnp.where(s2, 1.0, 0.0) * cents_sel[2])
    return centroid, gm


def _tc_kmeans_body(idxp_ref, grp_ref, emb_ref, cent_ref, len_ref,
                    rows_v, sem):
    copies = [
        pltpu.make_async_copy(emb_ref.at[pl.ds(idxp_ref[j] * 128, 128)],
                              rows_v.at[j], sem)
        for j in range(_TCA)
    ]
    for c in copies:
        c.start()
    for c in copies:
        c.wait()
    pmask = lax.broadcasted_iota(jnp.int32, (32, 1), 0) < _TCA
    grp = grp_ref[...]
    gath = rows_v[...]
    data = jnp.zeros((32, _DIM), jnp.float32)
    for g in range(4):
        data = data + jnp.where((grp == g) & pmask,
                                gath[:, g * _DIM:(g + 1) * _DIM], 0.0)
    centroid, gm = _kmeans_centroid(data, pmask)
    r = lax.broadcasted_iota(jnp.int32, (_DIM, _DIM), 0)
    c = lax.broadcasted_iota(jnp.int32, (_DIM, _DIM), 1)
    eye = (r == c).astype(jnp.float32)
    cent_col = lax.dot_general(eye, centroid, (((1,), (1,)), ((), ())))
    cent_ref[...] = lax.dot_general(cent_col, jnp.ones((1, 128), jnp.float32),
                                    (((1,), (0,)), ((), ())))
    len_ref[0, 0] = gm.astype(jnp.int32)


_tc_kmeans = pl.pallas_call(
    _tc_kmeans_body,
    in_specs=[
        pl.BlockSpec(memory_space=pltpu.SMEM),
        pl.BlockSpec(memory_space=pltpu.VMEM),
        pl.BlockSpec(memory_space=pl.ANY),
    ],
    out_shape=(
        jax.ShapeDtypeStruct((_DIM, 128), jnp.float32),
        jax.ShapeDtypeStruct((1, 1), jnp.int32),
    ),
    out_specs=(
        pl.BlockSpec(memory_space=pltpu.VMEM),
        pl.BlockSpec(memory_space=pltpu.SMEM),
    ),
    scratch_shapes=[
        pltpu.VMEM((32, 128), jnp.float32),
        pltpu.SemaphoreType.DMA,
    ],
)


@functools.cache
def _sc_scan_fn():
    mesh = plsc.VectorSubcoreMesh(core_axis_name="c", subcore_axis_name="s")

    @functools.partial(
        pl.kernel,
        out_type=(
            jax.ShapeDtypeStruct((_NW, 128), jnp.float32),
            jax.ShapeDtypeStruct((_NW, 128), jnp.int32),
        ),
        mesh=mesh,
        compiler_params=pltpu.CompilerParams(needs_layout_passes=False),
        scratch_types=[
            pltpu.VMEM((_CR * _DIM,), jnp.float32),
            pltpu.VMEM((_CR * _DIM,), jnp.float32),
            pltpu.VMEM((_DIM, 128), jnp.float32),
            pltpu.VMEM((128,), jnp.float32),
            pltpu.VMEM((128,), jnp.int32),
            pltpu.SemaphoreType.DMA,
            pltpu.SemaphoreType.DMA,
        ],
    )
    def _sc_scan(cent_hbm, emb_hbm, omin_hbm, orow_hbm,
                 buf_a, buf_b, cent_v, ovmin, ovrow, sem_a, sem_b):
        wid = lax.axis_index("s") * 2 + lax.axis_index("c")
        base = wid * _T16
        count = jnp.where(wid == _NW - 1, _TLAST, _T16)
        lane = lax.broadcasted_iota(jnp.int32, (16,), 0)

        pltpu.sync_copy(cent_hbm, cent_v)
        cvecs = [cent_v[d, pl.ds(0, 16)] for d in range(_DIM)]

        def chunk_start(cid):
            return base + jnp.minimum(cid * _CR, count - _CR)

        def copy(cid, buf, sem):
            return pltpu.make_async_copy(
                emb_hbm.at[pl.ds(chunk_start(cid) * _DIM, _CR * _DIM)],
                buf, sem)

        def compute(cid, buf, carry):
            st = chunk_start(cid)

            def jbody(j, carry2):
                minvec, rowb = carry2
                rows = st + j * 16 + lane
                fbase = (j * 16 + lane) * _DIM
                accs = [jnp.zeros((16,), jnp.float32) for _ in range(4)]
                for d in range(_DIM):
                    v = plsc.load_gather(buf, [fbase + d])
                    t = v - cvecs[d]
                    accs[d % 4] = accs[d % 4] + t * t
                acc = (accs[0] + accs[1]) + (accs[2] + accs[3])
                acc = jnp.where(rows < _NSCAN, acc, _BIG)
                upd = acc < minvec
                minvec = jnp.where(upd, acc, minvec)
                rowb = jnp.where(upd, rows, rowb)
                return (minvec, rowb)

            return lax.fori_loop(0, _GRP, jbody, carry)

        copy(0, buf_a, sem_a).start()

        def cbody(c2, carry):
            a = 2 * c2
            copy(a + 1, buf_b, sem_b).start()
            copy(a, buf_a, sem_a).wait()
            carry = compute(a, buf_a, carry)
            copy(jnp.minimum(a + 2, _NCH - 1), buf_a, sem_a).start()
            copy(a + 1, buf_b, sem_b).wait()
            carry = compute(a + 1, buf_b, carry)
            return carry

        minvec0 = jnp.full((16,), _BIG, jnp.float32)
        rowb0 = jnp.zeros((16,), jnp.int32)
        minvec, rowb = lax.fori_loop(0, _NCH // 2, cbody, (minvec0, rowb0))
        copy(_NCH - 1, buf_a, sem_a).wait()

        ovmin[pl.ds(0, 16)] = minvec
        ovrow[pl.ds(0, 16)] = rowb
        for t in range(1, 8):
            ovmin[pl.ds(t * 16, 16)] = jnp.full((16,), _BIG, jnp.float32)
            ovrow[pl.ds(t * 16, 16)] = jnp.full((16,), _VOCAB, jnp.int32)
        pltpu.sync_copy(ovmin, omin_hbm.at[wid])
        pltpu.sync_copy(ovrow, orow_hbm.at[wid])

    return _sc_scan


def _tc_merge_body(omin_ref, orow_ref, idx_ref, dist_ref):
    m = omin_ref[...]
    r = orow_ref[...]
    gmin = jnp.min(m)
    idx_ref[0, 0] = jnp.min(jnp.where(m == gmin, r, _VOCAB))
    dist_ref[0, 0] = jnp.sqrt(gmin)


_tc_merge = pl.pallas_call(
    _tc_merge_body,
    out_shape=(
        jax.ShapeDtypeStruct((1, 1), jnp.int32),
        jax.ShapeDtypeStruct((1, 1), jnp.float32),
    ),
    out_specs=(
        pl.BlockSpec(memory_space=pltpu.SMEM),
        pl.BlockSpec(memory_space=pltpu.SMEM),
    ),
)


def kernel(embeddings, good_idx, bad_idx):
    idx = jnp.concatenate([
        good_idx.astype(jnp.int32),
        bad_idx.astype(jnp.int32),
        jnp.zeros((32 - _TCA,), jnp.int32),
    ])
    embf = embeddings.reshape(_VOCAB * _DIM)
    idx_p = idx // 4
    grp = (idx % 4).reshape(32, 1)
    cent_b, clue_len = _tc_kmeans(idx_p, grp, embf)
    omin, orow = _sc_scan_fn()(cent_b, embf)
    clue_idx, min_dist = _tc_merge(omin, orow)
    return clue_idx[0, 0], clue_len[0, 0], min_dist[0, 0]


# speedup vs baseline: 1.4540x; 1.4540x over previous
import functools

import jax
import jax.numpy as jnp
from jax import lax
from jax.experimental import pallas as pl
from jax.experimental.pallas import tpu as pltpu
from jax.experimental.pallas import tpu_sc as plsc

_TCA = 25
_GCA = 9
_KS = (24, 23, 22)
_ITERS = 10
_MAXK = 24
_VOCAB = 1000000
_DIM = 32
_NSCAN = _VOCAB - 1
_NW = 32
_T16 = 31248
_TLAST = _VOCAB - (_NW - 1) * _T16
_CR = 1024
_NCH = 32
_GRP = _CR // 16
_BIG = 3.0e38


def _kmeans_run(data, pmask, k):
    jj = lax.broadcasted_iota(jnp.int32, (1, _MAXK), 1)
    kmask = jj < k
    cents0 = data[:_MAXK]
    ones_col = jnp.ones((32, 1), jnp.float32)

    def labels_of(cents):
        diff = data[:, None, :] - cents[None, :, :]
        d2 = jnp.sum(diff * diff, axis=-1)
        d2 = jnp.where(kmask, d2, _BIG)
        m = jnp.min(d2, axis=1, keepdims=True)
        return jnp.min(jnp.where(d2 == m, jj, _MAXK), axis=1, keepdims=True)

    def body(_, cents):
        lab = labels_of(cents)
        onehot = ((lab == jj) & pmask).astype(jnp.float32)
        counts = lax.dot_general(onehot, ones_col,
                                 (((0,), (0,)), ((), ())))
        sums = lax.dot_general(onehot, data,
                               (((0,), (0,)), ((), ())))
        newc = sums / jnp.maximum(counts, 1.0)
        return jnp.where(counts > 0, newc, cents)

    cents = lax.fori_loop(0, _ITERS, body, cents0)
    lab = labels_of(cents)
    onehot = ((lab == jj) & pmask).astype(jnp.float32)
    ii = lax.broadcasted_iota(jnp.int32, (32, 1), 0)
    good = jnp.sum(onehot * (ii < _GCA), axis=0, keepdims=True)
    bad = jnp.sum(onehot * ((ii >= _GCA) & pmask), axis=0, keepdims=True)
    sizes = jnp.where((bad == 0.0) & (good > 0.0), good, 0.0)
    return sizes, cents


def _kmeans_centroid(data, pmask):
    jj = lax.broadcasted_iota(jnp.int32, (1, _MAXK), 1)
    ms, cents_sel = [], []
    for k in _KS:
        sizes, cents = _kmeans_run(data, pmask, k)
        m = jnp.max(sizes)
        arg = jnp.min(jnp.where(sizes == m, jj, _MAXK))
        oh = (jj == arg).astype(jnp.float32)
        csel = lax.dot_general(oh, cents, (((1,), (0,)), ((), ())))
        ms.append(m)
        cents_sel.append(csel)

    gm = jnp.maximum(jnp.maximum(ms[0], ms[1]), ms[2])
    s0 = ms[0] == gm
    s1 = (ms[1] == gm) & (~s0)
    s2 = (ms[2] == gm) & (~s0) & (~s1)
    centroid = (jnp.where(s0, 1.0, 0.0) * cents_sel[0]
                + jnp.where(s1, 1.0, 0.0) * cents_sel[1]
                + jnp.where(s2, 1.0, 0.0) * cents_sel[2])
    return centroid, gm


def _tc_kmeans_body(idxp_ref, grp_ref, emb_ref, cent_ref, len_ref,
                    rows_v, sem):
    copies = [
        pltpu.make_async_copy(emb_ref.at[pl.ds(idxp_ref[j] * 128, 128)],
                              rows_v.at[j], sem)
        for j in range(_TCA)
    ]
    for c in copies:
        c.start()
    for c in copies:
        c.wait()
    pmask = lax.broadcasted_iota(jnp.int32, (32, 1), 0) < _TCA
    grp = grp_ref[...]
    gath = rows_v[...]
    data = jnp.zeros((32, _DIM), jnp.float32)
    for g in range(4):
        data = data + jnp.where((grp == g) & pmask,
                                gath[:, g * _DIM:(g + 1) * _DIM], 0.0)
    centroid, gm = _kmeans_centroid(data, pmask)
    dd = lax.broadcasted_iota(jnp.int32, (_DIM, 128), 0)
    ll = lax.broadcasted_iota(jnp.int32, (_DIM, 128), 1)
    rotidx = lax.rem(dd + ll, _DIM)
    crot = jnp.zeros((_DIM, 128), jnp.float32)
    for k in range(_DIM):
        crot = crot + centroid[0, k] * (rotidx == k).astype(jnp.float32)
    cent_ref[...] = crot
    len_ref[0, 0] = gm.astype(jnp.int32)


_tc_kmeans = pl.pallas_call(
    _tc_kmeans_body,
    in_specs=[
        pl.BlockSpec(memory_space=pltpu.SMEM),
        pl.BlockSpec(memory_space=pltpu.VMEM),
        pl.BlockSpec(memory_space=pl.ANY),
    ],
    out_shape=(
        jax.ShapeDtypeStruct((_DIM, 128), jnp.float32),
        jax.ShapeDtypeStruct((1, 1), jnp.int32),
    ),
    out_specs=(
        pl.BlockSpec(memory_space=pltpu.VMEM),
        pl.BlockSpec(memory_space=pltpu.SMEM),
    ),
    scratch_shapes=[
        pltpu.VMEM((32, 128), jnp.float32),
        pltpu.SemaphoreType.DMA,
    ],
)


@functools.cache
def _sc_scan_fn():
    mesh = plsc.VectorSubcoreMesh(core_axis_name="c", subcore_axis_name="s")

    @functools.partial(
        pl.kernel,
        out_type=(
            jax.ShapeDtypeStruct((_NW, 128), jnp.float32),
            jax.ShapeDtypeStruct((_NW, 128), jnp.int32),
        ),
        mesh=mesh,
        compiler_params=pltpu.CompilerParams(needs_layout_passes=False),
        scratch_types=[
            pltpu.VMEM((_CR * _DIM,), jnp.float32),
            pltpu.VMEM((_CR * _DIM,), jnp.float32),
            pltpu.VMEM((_DIM, 128), jnp.float32),
            pltpu.VMEM((128,), jnp.float32),
            pltpu.VMEM((128,), jnp.int32),
            pltpu.SemaphoreType.DMA,
            pltpu.SemaphoreType.DMA,
        ],
    )
    def _sc_scan(cent_hbm, emb_hbm, omin_hbm, orow_hbm,
                 buf_a, buf_b, cent_v, ovmin, ovrow, sem_a, sem_b):
        wid = lax.axis_index("s") * 2 + lax.axis_index("c")
        base = wid * _T16
        count = jnp.where(wid == _NW - 1, _TLAST, _T16)
        lane = lax.broadcasted_iota(jnp.int32, (16,), 0)
        lane33 = lane * (_DIM + 1)

        pltpu.sync_copy(cent_hbm, cent_v)
        cvecs = [cent_v[d, pl.ds(0, 16)] for d in range(_DIM)]

        def chunk_start(cid):
            return base + jnp.minimum(cid * _CR, count - _CR)

        def copy(cid, buf, sem):
            return pltpu.make_async_copy(
                emb_hbm.at[pl.ds(chunk_start(cid) * _DIM, _CR * _DIM)],
                buf, sem)

        def compute(cid, buf, carry):
            st = chunk_start(cid)

            def jbody(j, carry2):
                minvec, rowb = carry2
                rows = st + j * 16 + lane
                fbl = j * (16 * _DIM) + lane33
                accs = [jnp.zeros((16,), jnp.float32) for _ in range(4)]
                for d in range(_DIM):
                    if d <= 32 - 16:
                        idx = fbl + d
                    else:
                        idx = fbl + jnp.where(lane >= (_DIM - d),
                                              d - _DIM, d)
                    v = plsc.load_gather(buf, [idx])
                    t = v - cvecs[d]
                    accs[d % 4] = accs[d % 4] + t * t
                acc = (accs[0] + accs[1]) + (accs[2] + accs[3])
                acc = jnp.where(rows < _NSCAN, acc, _BIG)
                upd = acc < minvec
                minvec = jnp.where(upd, acc, minvec)
                rowb = jnp.where(upd, rows, rowb)
                return (minvec, rowb)

            return lax.fori_loop(0, _GRP, jbody, carry)

        copy(0, buf_a, sem_a).start()

        def cbody(c2, carry):
            a = 2 * c2
            copy(a + 1, buf_b, sem_b).start()
            copy(a, buf_a, sem_a).wait()
            carry = compute(a, buf_a, carry)
            copy(jnp.minimum(a + 2, _NCH - 1), buf_a, sem_a).start()
            copy(a + 1, buf_b, sem_b).wait()
            carry = compute(a + 1, buf_b, carry)
            return carry

        minvec0 = jnp.full((16,), _BIG, jnp.float32)
        rowb0 = jnp.zeros((16,), jnp.int32)
        minvec, rowb = lax.fori_loop(0, _NCH // 2, cbody, (minvec0, rowb0))
        copy(_NCH - 1, buf_a, sem_a).wait()

        ovmin[pl.ds(0, 16)] = minvec
        ovrow[pl.ds(0, 16)] = rowb
        for t in range(1, 8):
            ovmin[pl.ds(t * 16, 16)] = jnp.full((16,), _BIG, jnp.float32)
            ovrow[pl.ds(t * 16, 16)] = jnp.full((16,), _VOCAB, jnp.int32)
        pltpu.sync_copy(ovmin, omin_hbm.at[wid])
        pltpu.sync_copy(ovrow, orow_hbm.at[wid])

    return _sc_scan


def _tc_merge_body(omin_ref, orow_ref, idx_ref, dist_ref):
    m = omin_ref[...]
    r = orow_ref[...]
    gmin = jnp.min(m)
    idx_ref[0, 0] = jnp.min(jnp.where(m == gmin, r, _VOCAB))
    dist_ref[0, 0] = jnp.sqrt(gmin)


_tc_merge = pl.pallas_call(
    _tc_merge_body,
    out_shape=(
        jax.ShapeDtypeStruct((1, 1), jnp.int32),
        jax.ShapeDtypeStruct((1, 1), jnp.float32),
    ),
    out_specs=(
        pl.BlockSpec(memory_space=pltpu.SMEM),
        pl.BlockSpec(memory_space=pltpu.SMEM),
    ),
)


def kernel(embeddings, good_idx, bad_idx):
    idx = jnp.concatenate([
        good_idx.astype(jnp.int32),
        bad_idx.astype(jnp.int32),
        jnp.zeros((32 - _TCA,), jnp.int32),
    ])
    embf = embeddings.reshape(_VOCAB * _DIM)
    idx_p = idx // 4
    grp = (idx % 4).reshape(32, 1)
    cent_b, clue_len = _tc_kmeans(idx_p, grp, embf)
    omin, orow = _sc_scan_fn()(cent_b, embf)
    clue_idx, min_dist = _tc_merge(omin, orow)
    return clue_idx[0, 0], clue_len[0, 0], min_dist[0, 0]
